# baseline (speedup 1.0000x reference)
import functools

import jax
import jax.numpy as jnp
from jax import lax
from jax.experimental import pallas as pl
from jax.experimental.pallas import tpu as pltpu
from jax.experimental.pallas import tpu_sc as plsc

VOCAB = 1000000
EMBED = 64

_NC = 2
_NS = 16
_NW = _NC * _NS

_C = 128


def _gather_kernel_body(u_per_w, idxT_hbm, table_hbm, out_hbm,
                        idx_v0, idx_v1, rows_v0, rows_v1,
                        isem0, isem1, gsem0, gsem1, osem0, osem1):
    wid = lax.axis_index("s") * _NC + lax.axis_index("c")
    ubase = wid * u_per_w
    idx_v = (idx_v0, idx_v1)
    rows_v = (rows_v0, rows_v1)
    isem = (isem0, isem1)
    gsem = (gsem0, gsem1)
    osem = (osem0, osem1)

    def unit_sc(a):
        return a >> 7, (a & 127) * _C

    def fire_idx(a, b):
        s, c = unit_sc(a)
        pltpu.async_copy(idxT_hbm.at[s, pl.ds(c, _C)], idx_v[b], isem[b])

    def drain_idx(b):
        pltpu.make_async_copy(
            idxT_hbm.at[0, pl.ds(0, _C)], idx_v[b], isem[b]).wait()

    def fire_gather(b):
        pltpu.async_copy(table_hbm.at[idx_v[b]], rows_v[b], gsem[b])

    def drain_gather(b):
        pltpu.make_async_copy(
            table_hbm.at[pl.ds(0, _C)], rows_v[b], gsem[b]).wait()

    def fire_out(a, b):
        s, c = unit_sc(a)
        pltpu.async_copy(rows_v[b], out_hbm.at[s, pl.ds(c, _C)], osem[b])

    def drain_out(b):
        pltpu.make_async_copy(
            rows_v[b], out_hbm.at[0, pl.ds(0, _C)], osem[b]).wait()

    fire_idx(ubase + 0, 0)
    fire_idx(ubase + 1, 1)
    drain_idx(0)
    fire_gather(0)

    def step(u, with_drain_out, with_fire_idx):
        b = u % 2
        nb = 1 - b
        a = ubase + u
        drain_idx(b)
        drain_gather(nb)
        fire_gather(b)
        if with_fire_idx:
            fire_idx(a + 1, nb)
        if with_drain_out:
            drain_out(nb)
        fire_out(a - 1, nb)

    step(1, False, True)
    step(2, False, True)

    def pair(m, carry):
        for t in range(2):
            u = 3 + 2 * m + t
            b = (3 + t) % 2
            nb = 1 - b
            a = ubase + u
            drain_idx(b)
            drain_gather(nb)
            fire_gather(b)
            fire_idx(a + 1, nb)
            drain_out(nb)
            fire_out(a - 1, nb)
        return carry

    lax.fori_loop(0, (u_per_w - 4) // 2, pair, 0)

    step(u_per_w - 1, True, False)

    bl = (u_per_w - 1) % 2
    drain_gather(bl)
    drain_out(bl)
    fire_out(ubase + u_per_w - 1, bl)
    drain_out(bl)
    drain_out(1 - bl)


def _tc_transpose_body(x_ref, o_ref):
    x = x_ref[0]
    y = x.T.reshape(8, 8, 16, 128)
    o_ref[0] = jnp.transpose(y, (0, 2, 1, 3))


def kernel(idx, table):
    nb, ns = idx.shape
    idxT = idx.T.astype(jnp.int32)
    units = ns * (nb // _C)
    u_per_w = units // _NW

    mesh = plsc.VectorSubcoreMesh(core_axis_name="c", subcore_axis_name="s")
    k = functools.partial(
        pl.kernel,
        mesh=mesh,
        out_type=jax.ShapeDtypeStruct((ns, nb, EMBED), jnp.float32),
        scratch_types=[
            pltpu.VMEM((_C,), jnp.int32),
            pltpu.VMEM((_C,), jnp.int32),
            pltpu.VMEM((_C, EMBED), jnp.float32),
            pltpu.VMEM((_C, EMBED), jnp.float32),
            pltpu.SemaphoreType.DMA,
            pltpu.SemaphoreType.DMA,
            pltpu.SemaphoreType.DMA,
            pltpu.SemaphoreType.DMA,
            pltpu.SemaphoreType.DMA,
            pltpu.SemaphoreType.DMA,
        ],
        compiler_params=pltpu.CompilerParams(use_tc_tiling_on_sc=False),
    )(functools.partial(_gather_kernel_body, u_per_w))

    out2 = k(idxT, table)

    nbt = nb // 128
    out5 = pl.pallas_call(
        _tc_transpose_body,
        grid=(ns, nb // 2048),
        in_specs=[pl.BlockSpec((1, 2048, EMBED), lambda s, j: (s, j, 0))],
        out_specs=pl.BlockSpec((1, EMBED // 8, 16, 8, 128),
                               lambda s, j: (s, 0, j, 0, 0)),
        out_shape=jax.ShapeDtypeStruct((ns, EMBED // 8, nbt, 8, 128),
                                       jnp.float32),
    )(out2)

    return out5.transpose(2, 4, 0, 1, 3).reshape(nb, ns, EMBED)


# speedup vs baseline: 1.0650x; 1.0650x over previous
import functools

import jax
import jax.numpy as jnp
from jax import lax
from jax.experimental import pallas as pl
from jax.experimental.pallas import tpu as pltpu
from jax.experimental.pallas import tpu_sc as plsc

VOCAB = 1000000
EMBED = 64

_NC = 2
_NS = 16
_NW = _NC * _NS

_C = 512
_CB = 16384 // _C


def _gather_kernel_body(u_per_w, idxT_hbm, table_hbm, out_hbm,
                        idx_v0, idx_v1, rows_v0, rows_v1,
                        isem0, isem1, gsem0, gsem1, osem0, osem1):
    wid = lax.axis_index("s") * _NC + lax.axis_index("c")
    ubase = wid * u_per_w
    idx_v = (idx_v0, idx_v1)
    rows_v = (rows_v0, rows_v1)
    isem = (isem0, isem1)
    gsem = (gsem0, gsem1)
    osem = (osem0, osem1)

    def unit_sc(a):
        return a >> 5, (a & (_CB - 1)) * _C

    def fire_idx(a, b):
        s, c = unit_sc(a)
        pltpu.async_copy(idxT_hbm.at[s, pl.ds(c, _C)], idx_v[b], isem[b])

    def drain_idx(b):
        pltpu.make_async_copy(
            idxT_hbm.at[0, pl.ds(0, _C)], idx_v[b], isem[b]).wait()

    def fire_gather(b):
        pltpu.async_copy(table_hbm.at[idx_v[b]], rows_v[b], gsem[b])

    def drain_gather(b):
        pltpu.make_async_copy(
            table_hbm.at[pl.ds(0, _C)], rows_v[b], gsem[b]).wait()

    def fire_out(a, b):
        s, c = unit_sc(a)
        pltpu.async_copy(rows_v[b], out_hbm.at[s, pl.ds(c, _C)], osem[b])

    def drain_out(b):
        pltpu.make_async_copy(
            rows_v[b], out_hbm.at[0, pl.ds(0, _C)], osem[b]).wait()

    fire_idx(ubase + 0, 0)
    fire_idx(ubase + 1, 1)
    drain_idx(0)
    fire_gather(0)

    def step(u, with_drain_out, with_fire_idx):
        b = u % 2
        nb = 1 - b
        a = ubase + u
        drain_idx(b)
        drain_gather(nb)
        fire_gather(b)
        if with_fire_idx:
            fire_idx(a + 1, nb)
        if with_drain_out:
            drain_out(nb)
        fire_out(a - 1, nb)

    step(1, False, True)
    step(2, False, True)

    def pair(m, carry):
        for t in range(2):
            u = 3 + 2 * m + t
            b = (3 + t) % 2
            nb = 1 - b
            a = ubase + u
            drain_idx(b)
            drain_gather(nb)
            fire_gather(b)
            fire_idx(a + 1, nb)
            drain_out(nb)
            fire_out(a - 1, nb)
        return carry

    lax.fori_loop(0, (u_per_w - 4) // 2, pair, 0)

    step(u_per_w - 1, True, False)

    bl = (u_per_w - 1) % 2
    drain_gather(bl)
    drain_out(bl)
    fire_out(ubase + u_per_w - 1, bl)
    drain_out(bl)
    drain_out(1 - bl)


def _tc_transpose_body(x_ref, o_ref):
    o_ref[0] = x_ref[0].T


def kernel(idx, table):
    nb, ns = idx.shape
    idxT = idx.T.astype(jnp.int32)
    units = ns * (nb // _C)
    u_per_w = units // _NW

    mesh = plsc.VectorSubcoreMesh(core_axis_name="c", subcore_axis_name="s")
    k = functools.partial(
        pl.kernel,
        mesh=mesh,
        out_type=jax.ShapeDtypeStruct((ns, nb, EMBED), jnp.float32),
        scratch_types=[
            pltpu.VMEM((_C,), jnp.int32),
            pltpu.VMEM((_C,), jnp.int32),
            pltpu.VMEM((_C, EMBED), jnp.float32),
            pltpu.VMEM((_C, EMBED), jnp.float32),
            pltpu.SemaphoreType.DMA,
            pltpu.SemaphoreType.DMA,
            pltpu.SemaphoreType.DMA,
            pltpu.SemaphoreType.DMA,
            pltpu.SemaphoreType.DMA,
            pltpu.SemaphoreType.DMA,
        ],
        compiler_params=pltpu.CompilerParams(use_tc_tiling_on_sc=False),
    )(functools.partial(_gather_kernel_body, u_per_w))

    out2 = k(idxT, table)

    outT = pl.pallas_call(
        _tc_transpose_body,
        grid=(ns, nb // 2048),
        in_specs=[pl.BlockSpec((1, 2048, EMBED), lambda s, j: (s, j, 0))],
        out_specs=pl.BlockSpec((1, EMBED, 2048), lambda s, j: (s, 0, j)),
        out_shape=jax.ShapeDtypeStruct((ns, EMBED, nb), jnp.float32),
    )(out2)

    return outT.transpose(2, 0, 1)


# speedup vs baseline: 1.0662x; 1.0011x over previous
import functools

import jax
import jax.numpy as jnp
from jax import lax
from jax.experimental import pallas as pl
from jax.experimental.pallas import tpu as pltpu
from jax.experimental.pallas import tpu_sc as plsc

VOCAB = 1000000
EMBED = 64

_NC = 2
_NS = 16
_NW = _NC * _NS

_C = 512
_CB = 16384 // _C


def _gather_kernel_body(u_per_w, idxT_hbm, table_hbm, out_hbm,
                        idx_v0, idx_v1, rows_v0, rows_v1,
                        isem0, isem1, gsem0, gsem1, osem0, osem1):
    wid = lax.axis_index("s") * _NC + lax.axis_index("c")
    ubase = wid * u_per_w
    idx_v = (idx_v0, idx_v1)
    rows_v = (rows_v0, rows_v1)
    isem = (isem0, isem1)
    gsem = (gsem0, gsem1)
    osem = (osem0, osem1)

    def unit_sc(a):
        return a >> 5, (a & (_CB - 1)) * _C

    def fire_idx(a, b):
        s, c = unit_sc(a)
        pltpu.async_copy(idxT_hbm.at[s, pl.ds(c, _C)], idx_v[b], isem[b])

    def drain_idx(b):
        pltpu.make_async_copy(
            idxT_hbm.at[0, pl.ds(0, _C)], idx_v[b], isem[b]).wait()

    def fire_gather(b):
        pltpu.async_copy(table_hbm.at[idx_v[b]], rows_v[b], gsem[b])

    def drain_gather(b):
        pltpu.make_async_copy(
            table_hbm.at[pl.ds(0, _C)], rows_v[b], gsem[b]).wait()

    def fire_out(a, b):
        s, c = unit_sc(a)
        pltpu.async_copy(rows_v[b], out_hbm.at[s, pl.ds(c, _C)], osem[b])

    def drain_out(b):
        pltpu.make_async_copy(
            rows_v[b], out_hbm.at[0, pl.ds(0, _C)], osem[b]).wait()

    fire_idx(ubase + 0, 0)
    fire_idx(ubase + 1, 1)
    drain_idx(0)
    fire_gather(0)

    def step(u, with_drain_out, with_fire_idx):
        b = u % 2
        nb = 1 - b
        a = ubase + u
        drain_idx(b)
        drain_gather(nb)
        fire_gather(b)
        if with_fire_idx:
            fire_idx(a + 1, nb)
        if with_drain_out:
            drain_out(nb)
        fire_out(a - 1, nb)

    step(1, False, True)
    step(2, False, True)

    def pair(m, carry):
        for t in range(2):
            u = 3 + 2 * m + t
            b = (3 + t) % 2
            nb = 1 - b
            a = ubase + u
            drain_idx(b)
            drain_gather(nb)
            fire_gather(b)
            fire_idx(a + 1, nb)
            drain_out(nb)
            fire_out(a - 1, nb)
        return carry

    lax.fori_loop(0, (u_per_w - 4) // 2, pair, 0)

    step(u_per_w - 1, True, False)

    bl = (u_per_w - 1) % 2
    drain_gather(bl)
    drain_out(bl)
    fire_out(ubase + u_per_w - 1, bl)
    drain_out(bl)
    drain_out(1 - bl)


def _tc_transpose_body(x_ref, o_ref):
    eye = jnp.eye(EMBED, dtype=jnp.float32)
    o_ref[0] = jax.lax.dot_general(
        eye, x_ref[0], (((1,), (1,)), ((), ())),
        preferred_element_type=jnp.float32)


def kernel(idx, table):
    nb, ns = idx.shape
    idxT = idx.T.astype(jnp.int32)
    units = ns * (nb // _C)
    u_per_w = units // _NW

    mesh = plsc.VectorSubcoreMesh(core_axis_name="c", subcore_axis_name="s")
    k = functools.partial(
        pl.kernel,
        mesh=mesh,
        out_type=jax.ShapeDtypeStruct((ns, nb, EMBED), jnp.float32),
        scratch_types=[
            pltpu.VMEM((_C,), jnp.int32),
            pltpu.VMEM((_C,), jnp.int32),
            pltpu.VMEM((_C, EMBED), jnp.float32),
            pltpu.VMEM((_C, EMBED), jnp.float32),
            pltpu.SemaphoreType.DMA,
            pltpu.SemaphoreType.DMA,
            pltpu.SemaphoreType.DMA,
            pltpu.SemaphoreType.DMA,
            pltpu.SemaphoreType.DMA,
            pltpu.SemaphoreType.DMA,
        ],
        compiler_params=pltpu.CompilerParams(use_tc_tiling_on_sc=False),
    )(functools.partial(_gather_kernel_body, u_per_w))

    out2 = k(idxT, table)

    outT = pl.pallas_call(
        _tc_transpose_body,
        grid=(ns, nb // 2048),
        in_specs=[pl.BlockSpec((1, 2048, EMBED), lambda s, j: (s, j, 0))],
        out_specs=pl.BlockSpec((1, EMBED, 2048), lambda s, j: (s, 0, j)),
        out_shape=jax.ShapeDtypeStruct((ns, EMBED, nb), jnp.float32),
    )(out2)

    return outT.transpose(2, 0, 1)


# speedup vs baseline: 1.2020x; 1.1273x over previous
import functools

import jax
import jax.numpy as jnp
from jax import lax
from jax.experimental import pallas as pl
from jax.experimental.pallas import tpu as pltpu
from jax.experimental.pallas import tpu_sc as plsc

VOCAB = 1000000
EMBED = 64

_NC = 2
_NS = 16
_NW = _NC * _NS

_C = 512
_CB = 16384 // _C


def _gather_kernel_body(u_per_w, idxT_hbm, table_hbm, out_hbm,
                        idx_v0, idx_v1, rows_v0, rows_v1,
                        isem0, isem1, gsem0, gsem1, osem0, osem1):
    wid = lax.axis_index("s") * _NC + lax.axis_index("c")
    ubase = wid * u_per_w
    idx_v = (idx_v0, idx_v1)
    rows_v = (rows_v0, rows_v1)
    isem = (isem0, isem1)
    gsem = (gsem0, gsem1)
    osem = (osem0, osem1)

    def unit_sc(a):
        return a >> 5, (a & (_CB - 1)) * _C

    def fire_idx(a, b):
        s, c = unit_sc(a)
        pltpu.async_copy(idxT_hbm.at[s, pl.ds(c, _C)], idx_v[b], isem[b])

    def drain_idx(b):
        pltpu.make_async_copy(
            idxT_hbm.at[0, pl.ds(0, _C)], idx_v[b], isem[b]).wait()

    def fire_gather(b):
        pltpu.async_copy(table_hbm.at[idx_v[b]], rows_v[b], gsem[b])

    def drain_gather(b):
        pltpu.make_async_copy(
            table_hbm.at[pl.ds(0, _C)], rows_v[b], gsem[b]).wait()

    def fire_out(a, b):
        s, c = unit_sc(a)
        pltpu.async_copy(rows_v[b], out_hbm.at[s, pl.ds(c, _C)], osem[b])

    def drain_out(b):
        pltpu.make_async_copy(
            rows_v[b], out_hbm.at[0, pl.ds(0, _C)], osem[b]).wait()

    fire_idx(ubase + 0, 0)
    fire_idx(ubase + 1, 1)
    drain_idx(0)
    fire_gather(0)

    def step(u, with_drain_out, with_fire_idx):
        b = u % 2
        nb = 1 - b
        a = ubase + u
        drain_idx(b)
        drain_gather(nb)
        fire_gather(b)
        if with_fire_idx:
            fire_idx(a + 1, nb)
        if with_drain_out:
            drain_out(nb)
        fire_out(a - 1, nb)

    step(1, False, True)
    step(2, False, True)

    def pair(m, carry):
        for t in range(2):
            u = 3 + 2 * m + t
            b = (3 + t) % 2
            nb = 1 - b
            a = ubase + u
            drain_idx(b)
            drain_gather(nb)
            fire_gather(b)
            fire_idx(a + 1, nb)
            drain_out(nb)
            fire_out(a - 1, nb)
        return carry

    lax.fori_loop(0, (u_per_w - 4) // 2, pair, 0)

    step(u_per_w - 1, True, False)

    bl = (u_per_w - 1) % 2
    drain_gather(bl)
    drain_out(bl)
    fire_out(ubase + u_per_w - 1, bl)
    drain_out(bl)
    drain_out(1 - bl)


def _tc_transpose_body(x_ref, o_ref):
    o_ref[0] = x_ref[0].T


def kernel(idx, table):
    nb, ns = idx.shape
    idxT = idx.T.astype(jnp.int32)
    units = ns * (nb // _C)
    u_per_w = units // _NW

    mesh = plsc.VectorSubcoreMesh(core_axis_name="c", subcore_axis_name="s")
    k = functools.partial(
        pl.kernel,
        mesh=mesh,
        out_type=jax.ShapeDtypeStruct((ns, nb, EMBED), jnp.float32),
        scratch_types=[
            pltpu.VMEM((_C,), jnp.int32),
            pltpu.VMEM((_C,), jnp.int32),
            pltpu.VMEM((_C, EMBED), jnp.float32),
            pltpu.VMEM((_C, EMBED), jnp.float32),
            pltpu.SemaphoreType.DMA,
            pltpu.SemaphoreType.DMA,
            pltpu.SemaphoreType.DMA,
            pltpu.SemaphoreType.DMA,
            pltpu.SemaphoreType.DMA,
            pltpu.SemaphoreType.DMA,
        ],
        compiler_params=pltpu.CompilerParams(use_tc_tiling_on_sc=False),
    )(functools.partial(_gather_kernel_body, u_per_w))

    out2 = k(idxT, table)

    outT = pl.pallas_call(
        _tc_transpose_body,
        grid=(ns, nb // 8192),
        in_specs=[pl.BlockSpec((1, 8192, EMBED), lambda s, j: (s, j, 0))],
        out_specs=pl.BlockSpec((1, EMBED, 8192), lambda s, j: (s, 0, j)),
        out_shape=jax.ShapeDtypeStruct((ns, EMBED, nb), jnp.float32),
    )(out2)

    return outT.transpose(2, 0, 1)
